# baseline (speedup 1.0000x reference)
import jax
import jax.numpy as jnp
from jax import lax
from jax.experimental import pallas as pl
from jax.experimental.pallas import tpu as pltpu
from jax.experimental.pallas import tpu_sc as plsc

_B, _L, _D = 1024, 200, 64
_CH = 128
_info = plsc.get_sparse_core_info()
_NC = _info.num_cores
_NW = _info.num_cores * _info.num_subcores
_ROWS_W = _B * _L // _NW
_NCH = _ROWS_W // _CH
_NSUP = _NCH // 2


def _body(seq_hbm, seg_hbm, tok_hbm, segtab_hbm, pe_hbm, out_hbm,
          idx_v, sidx_v, pe2x, tok0, tok1, sg0, sg1,
          st0, st1, ss0, ss1, so0, so1):
    c = lax.axis_index("c")
    s = lax.axis_index("s")
    wid = s * _NC + c
    r0 = wid * _NCH

    pltpu.sync_copy(seq_hbm.at[wid], idx_v)
    pltpu.sync_copy(seg_hbm.at[wid], sidx_v)
    pltpu.sync_copy(pe_hbm, pe2x.at[pl.ds(0, _L)])
    pltpu.sync_copy(pe_hbm, pe2x.at[pl.ds(_L, _L)])

    def issue(ci, tok_buf, sg_buf, sem_t, sem_s):
        pltpu.async_copy(tok_hbm.at[idx_v.at[ci]], tok_buf, sem_t)
        pltpu.async_copy(segtab_hbm.at[sidx_v.at[ci]], sg_buf, sem_s)

    def wait_g(tok_buf, sg_buf, sem_t, sem_s):
        pltpu.make_async_copy(tok_hbm.at[idx_v.at[0]], tok_buf, sem_t).wait()
        pltpu.make_async_copy(segtab_hbm.at[sidx_v.at[0]], sg_buf, sem_s).wait()

    def wait_st(tok_buf, sem_o):
        pltpu.make_async_copy(tok_buf, out_hbm.at[pl.ds(0, _CH)], sem_o).wait()

    def add_and_store(ci, tok_buf, sg_buf, sem_o):
        p0 = lax.rem(ci * _CH, _L)

        def row(l, cr):
            for q in range(_D // 16):
                sl = pl.ds(q * 16, 16)
                tok_buf[l, sl] = (tok_buf[l, sl] + pe2x[p0 + l, sl]
                                  + sg_buf[l, sl])
            return cr

        lax.fori_loop(0, _CH, row, 0)
        pltpu.async_copy(tok_buf, out_hbm.at[pl.ds((r0 + ci) * _CH, _CH)],
                         sem_o)

    issue(0, tok0, sg0, st0, ss0)

    def super_iter(su, carry):
        ci = su * 2
        @pl.when(su >= 1)
        def _():
            wait_st(tok1, so1)
        issue(ci + 1, tok1, sg1, st1, ss1)
        wait_g(tok0, sg0, st0, ss0)
        add_and_store(ci, tok0, sg0, so0)
        wait_st(tok0, so0)
        @pl.when(su < _NSUP - 1)
        def _():
            issue(ci + 2, tok0, sg0, st0, ss0)
        wait_g(tok1, sg1, st1, ss1)
        add_and_store(ci + 1, tok1, sg1, so1)
        return carry

    lax.fori_loop(0, _NSUP, super_iter, 0)
    wait_st(tok1, so1)


def kernel(sequence, segment_label, token_table, seg_table, pe):
    b, l = sequence.shape
    v, d = token_table.shape
    seqf = sequence.astype(jnp.int32).reshape(_NW, _NCH, _CH)
    segf = segment_label.astype(jnp.int32).reshape(_NW, _NCH, _CH)
    pe2 = pe[0, :l, :]

    k = pl.kernel(
        _body,
        out_type=jax.ShapeDtypeStruct((b * l, d), jnp.float32),
        mesh=plsc.VectorSubcoreMesh(core_axis_name="c", subcore_axis_name="s"),
        compiler_params=pltpu.CompilerParams(use_tc_tiling_on_sc=False),
        scratch_types=[
            pltpu.VMEM((_NCH, _CH), jnp.int32),
            pltpu.VMEM((_NCH, _CH), jnp.int32),
            pltpu.VMEM((2 * _L, _D), jnp.float32),
            pltpu.VMEM((_CH, _D), jnp.float32),
            pltpu.VMEM((_CH, _D), jnp.float32),
            pltpu.VMEM((_CH, _D), jnp.float32),
            pltpu.VMEM((_CH, _D), jnp.float32),
            pltpu.SemaphoreType.DMA,
            pltpu.SemaphoreType.DMA,
            pltpu.SemaphoreType.DMA,
            pltpu.SemaphoreType.DMA,
            pltpu.SemaphoreType.DMA,
            pltpu.SemaphoreType.DMA,
        ],
    )
    out = k(seqf, segf, token_table, seg_table, pe2)
    return out.reshape(b, l, d)


# speedup vs baseline: 10.5515x; 10.5515x over previous
import jax
import jax.numpy as jnp
from jax import lax
from jax.experimental import pallas as pl
from jax.experimental.pallas import tpu as pltpu
from jax.experimental.pallas import tpu_sc as plsc

_B, _L, _D = 1024, 200, 64
_NSEG = 3
_CH = 128
_info = plsc.get_sparse_core_info()
_NC = _info.num_cores
_NW = _info.num_cores * _info.num_subcores
_ROWS_W = _B * _L // _NW
_NCH = _ROWS_W // _CH
_NSUP = _NCH // 2


def _body(seq_hbm, seg_hbm, tok_hbm, segtab_hbm, pe_hbm, out_hbm,
          idx_v, sidx_v, pe_v, segtab_v, build_v, peseg_sh,
          tok0, tok1, ps0, ps1,
          st0, st1, ss0, ss1, so0, so1):
    c = lax.axis_index("c")
    s = lax.axis_index("s")
    wid = s * _NC + c
    r0 = wid * _NCH

    pltpu.sync_copy(seq_hbm.at[wid], idx_v)
    pltpu.sync_copy(seg_hbm.at[wid], sidx_v)

    iota = lax.iota(jnp.int32, 16)

    def comb_chunk(ci, cr):
        base = ci * _CH
        for k in range(_CH // 16):
            sl = pl.ds(k * 16, 16)
            lrow = lax.rem(base + k * 16 + iota, _L)
            sidx_v[ci, sl] = lrow * _NSEG + sidx_v[ci, sl]
        return cr

    lax.fori_loop(0, _NCH, comb_chunk, 0)

    @pl.when(s == 0)
    def _():
        pltpu.sync_copy(pe_hbm, pe_v)
        pltpu.sync_copy(segtab_hbm, segtab_v)

        def build_row(l, cr):
            for sg in range(_NSEG):
                for q in range(_D // 16):
                    sl = pl.ds(q * 16, 16)
                    build_v[l * _NSEG + sg, sl] = (pe_v[l, sl]
                                                   + segtab_v[sg, sl])
            return cr

        lax.fori_loop(0, _L, build_row, 0)
        pltpu.sync_copy(build_v, peseg_sh)

    plsc.subcore_barrier()

    def issue(ci, tok_buf, ps_buf, sem_t, sem_s):
        pltpu.async_copy(tok_hbm.at[idx_v.at[ci]], tok_buf, sem_t)
        pltpu.async_copy(peseg_sh.at[sidx_v.at[ci]], ps_buf, sem_s)

    def wait_g(tok_buf, ps_buf, sem_t, sem_s):
        pltpu.make_async_copy(tok_hbm.at[idx_v.at[0]], tok_buf, sem_t).wait()
        pltpu.make_async_copy(peseg_sh.at[sidx_v.at[0]], ps_buf, sem_s).wait()

    def wait_st(tok_buf, sem_o):
        pltpu.make_async_copy(tok_buf, out_hbm.at[pl.ds(0, _CH)], sem_o).wait()

    def add_and_store(ci, tok_buf, ps_buf, sem_o):
        def row(l, cr):
            for q in range(_D // 16):
                sl = pl.ds(q * 16, 16)
                tok_buf[l, sl] = tok_buf[l, sl] + ps_buf[l, sl]
            return cr

        lax.fori_loop(0, _CH, row, 0)
        pltpu.async_copy(tok_buf, out_hbm.at[pl.ds((r0 + ci) * _CH, _CH)],
                         sem_o)

    issue(0, tok0, ps0, st0, ss0)

    def super_iter(su, carry):
        ci = su * 2
        @pl.when(su >= 1)
        def _():
            wait_st(tok1, so1)
        issue(ci + 1, tok1, ps1, st1, ss1)
        wait_g(tok0, ps0, st0, ss0)
        add_and_store(ci, tok0, ps0, so0)
        wait_st(tok0, so0)
        @pl.when(su < _NSUP - 1)
        def _():
            issue(ci + 2, tok0, ps0, st0, ss0)
        wait_g(tok1, ps1, st1, ss1)
        add_and_store(ci + 1, tok1, ps1, so1)
        return carry

    lax.fori_loop(0, _NSUP, super_iter, 0)
    wait_st(tok1, so1)


def kernel(sequence, segment_label, token_table, seg_table, pe):
    b, l = sequence.shape
    v, d = token_table.shape
    seqf = sequence.astype(jnp.int32).reshape(_NW, _NCH, _CH)
    segf = segment_label.astype(jnp.int32).reshape(_NW, _NCH, _CH)
    pe2 = pe[0, :l, :]

    k = pl.kernel(
        _body,
        out_type=jax.ShapeDtypeStruct((b * l, d), jnp.float32),
        mesh=plsc.VectorSubcoreMesh(core_axis_name="c", subcore_axis_name="s"),
        compiler_params=pltpu.CompilerParams(use_tc_tiling_on_sc=False),
        scratch_types=[
            pltpu.VMEM((_NCH, _CH), jnp.int32),
            pltpu.VMEM((_NCH, _CH), jnp.int32),
            pltpu.VMEM((_L, _D), jnp.float32),
            pltpu.VMEM((_NSEG, _D), jnp.float32),
            pltpu.VMEM((_L * _NSEG, _D), jnp.float32),
            pltpu.VMEM_SHARED((_L * _NSEG, _D), jnp.float32),
            pltpu.VMEM((_CH, _D), jnp.float32),
            pltpu.VMEM((_CH, _D), jnp.float32),
            pltpu.VMEM((_CH, _D), jnp.float32),
            pltpu.VMEM((_CH, _D), jnp.float32),
            pltpu.SemaphoreType.DMA,
            pltpu.SemaphoreType.DMA,
            pltpu.SemaphoreType.DMA,
            pltpu.SemaphoreType.DMA,
            pltpu.SemaphoreType.DMA,
            pltpu.SemaphoreType.DMA,
        ],
    )
    out = k(seqf, segf, token_table, seg_table, pe2)
    return out.reshape(b, l, d)


# speedup vs baseline: 10.6822x; 1.0124x over previous
import jax
import jax.numpy as jnp
from jax import lax
from jax.experimental import pallas as pl
from jax.experimental.pallas import tpu as pltpu
from jax.experimental.pallas import tpu_sc as plsc

_B, _L, _D = 1024, 200, 64
_NSEG = 3
_CH = 128
_NBUF = 5
_info = plsc.get_sparse_core_info()
_NC = _info.num_cores
_NW = _info.num_cores * _info.num_subcores
_ROWS_W = _B * _L // _NW
_NCH = _ROWS_W // _CH
_NSUP = _NCH // _NBUF
_LP = _L // _NBUF


def _body(seq_hbm, seg_hbm, tok_hbm, segtab_hbm, pe_hbm, out_hbm,
          idx_v, sidx_v, pe_v, segtab_v, peseg_sh, tok, ps, st, ss, so):
    c = lax.axis_index("c")
    s = lax.axis_index("s")
    wid = s * _NC + c
    r0 = wid * _NCH

    pltpu.sync_copy(seq_hbm.at[wid], idx_v)
    pltpu.sync_copy(seg_hbm.at[wid], sidx_v)

    iota = lax.iota(jnp.int32, 16)

    def comb_chunk(ci, cr):
        base = ci * _CH
        for k in range(_CH // 16):
            sl = pl.ds(k * 16, 16)
            lrow = lax.rem(base + k * 16 + iota, _L)
            sidx_v[ci, sl] = lrow * _NSEG + sidx_v[ci, sl]
        return cr

    lax.fori_loop(0, _NCH, comb_chunk, 0)

    @pl.when(s == 0)
    def _():
        pltpu.sync_copy(pe_hbm, pe_v)
        pltpu.sync_copy(segtab_hbm, segtab_v)
        for p in range(_NBUF):
            def build_row(l2, cr, p=p):
                l = p * _LP + l2
                for sg in range(_NSEG):
                    for q in range(_D // 16):
                        sl = pl.ds(q * 16, 16)
                        tok[0][l2 * _NSEG + sg, sl] = (pe_v[l, sl]
                                                       + segtab_v[sg, sl])
                return cr

            lax.fori_loop(0, _LP, build_row, 0)
            pltpu.sync_copy(tok[0].at[pl.ds(0, _LP * _NSEG)],
                            peseg_sh.at[pl.ds(p * _LP * _NSEG, _LP * _NSEG)])

    plsc.subcore_barrier()

    def issue(ci, k):
        pltpu.async_copy(tok_hbm.at[idx_v.at[ci]], tok[k], st[k])
        pltpu.async_copy(peseg_sh.at[sidx_v.at[ci]], ps[k], ss[k])

    def wait_g(k):
        pltpu.make_async_copy(tok_hbm.at[idx_v.at[0]], tok[k], st[k]).wait()
        pltpu.make_async_copy(peseg_sh.at[sidx_v.at[0]], ps[k], ss[k]).wait()

    def wait_st(k):
        pltpu.make_async_copy(tok[k], out_hbm.at[pl.ds(0, _CH)],
                              so[k]).wait()

    def add_and_store(ci, k):
        def row(l, cr):
            for q in range(_D // 16):
                sl = pl.ds(q * 16, 16)
                tok[k][l, sl] = tok[k][l, sl] + ps[k][l, sl]
            return cr

        lax.fori_loop(0, _CH, row, 0)
        pltpu.async_copy(tok[k], out_hbm.at[pl.ds((r0 + ci) * _CH, _CH)],
                         so[k])

    for k in range(_NBUF - 1):
        issue(k, k)

    def super_iter(su, carry):
        for k in range(_NBUF):
            ci = su * _NBUF + k
            prev = (k + _NBUF - 1) % _NBUF

            @pl.when(ci >= 1)
            def _(prev=prev):
                wait_st(prev)

            @pl.when(ci + _NBUF - 1 < _NCH)
            def _(ci=ci, prev=prev):
                issue(ci + _NBUF - 1, prev)

            wait_g(k)
            add_and_store(ci, k)
        return carry

    lax.fori_loop(0, _NSUP, super_iter, 0)
    wait_st((_NCH - 1) % _NBUF)


def kernel(sequence, segment_label, token_table, seg_table, pe):
    b, l = sequence.shape
    v, d = token_table.shape
    seqf = sequence.astype(jnp.int32).reshape(_NW, _NCH, _CH)
    segf = segment_label.astype(jnp.int32).reshape(_NW, _NCH, _CH)
    pe2 = pe[0, :l, :]

    row_buf = pltpu.VMEM((_CH, _D), jnp.float32)
    k = pl.kernel(
        _body,
        out_type=jax.ShapeDtypeStruct((b * l, d), jnp.float32),
        mesh=plsc.VectorSubcoreMesh(core_axis_name="c", subcore_axis_name="s"),
        compiler_params=pltpu.CompilerParams(use_tc_tiling_on_sc=False),
        scratch_types=[
            pltpu.VMEM((_NCH, _CH), jnp.int32),
            pltpu.VMEM((_NCH, _CH), jnp.int32),
            pltpu.VMEM((_L, _D), jnp.float32),
            pltpu.VMEM((_NSEG, _D), jnp.float32),
            pltpu.VMEM_SHARED((_L * _NSEG, _D), jnp.float32),
            [row_buf] * _NBUF,
            [row_buf] * _NBUF,
            [pltpu.SemaphoreType.DMA] * _NBUF,
            [pltpu.SemaphoreType.DMA] * _NBUF,
            [pltpu.SemaphoreType.DMA] * _NBUF,
        ],
    )
    out = k(seqf, segf, token_table, seg_table, pe2)
    return out.reshape(b, l, d)
